# baseline (speedup 1.0000x reference)
import jax
import jax.numpy as jnp
from jax import lax
from jax.experimental import pallas as pl
from jax.experimental.pallas import tpu as pltpu
from jax.experimental.pallas import tpu_sc as plsc

BATCH = 16384
LENGTH = 1024
WIDTH = 2 * LENGTH
NW = 32
RPW = BATCH // NW
CH = 2
NCH = RPW // CH
NGRP = WIDTH // 16


def _sc_body(x_hbm, a_hbm, b_hbm, o_hbm, xb, ob, av, bv,
             isem0, isem1, osem0, osem1):
    wid = lax.axis_index("s") * 2 + lax.axis_index("c")
    base_row = wid * RPW

    pltpu.sync_copy(a_hbm, av)
    pltpu.sync_copy(b_hbm, bv)

    lane = lax.iota(jnp.int32, 16)
    half = lane >> 1
    m4 = half & 3
    cpat = half + jnp.where(m4 == 1, 1, 0) - jnp.where(m4 == 2, 1, 0)
    re0 = lane & 1
    re1 = 1 - re0

    isems = (isem0, isem1)
    osems = (osem0, osem1)

    def in_copy(c, par):
        row0 = base_row + c * CH
        return pltpu.make_async_copy(
            x_hbm.at[pl.ds(row0, CH)], xb.at[par], isems[par])

    def out_copy(c, par):
        row0 = base_row + c * CH
        return pltpu.make_async_copy(
            ob.at[par], o_hbm.at[pl.ds(row0, CH)], osems[par])

    in_copy(0, 0).start()
    in_copy(1, 1).start()

    def chunk_pair(c2, carry):
        for par in (0, 1):
            c = 2 * c2 + par
            in_copy(c, par).wait()

            @pl.when(c >= 2)
            def _():
                out_copy(c - 2, par).wait()

            xbp = xb.at[par]
            obp = ob.at[par]

            def grp_body(gg, carry2):
                c0 = gg * 8
                cols = cpat + c0
                ocols = half + c0
                a = av[pl.ds(gg * 16, 16)]
                b = bv[pl.ds(gg * 16, 16)]
                for r in range(CH):
                    rows = jnp.full((16,), r, jnp.int32)
                    y = plsc.load_gather(xbp, [rows, cols, re0])
                    ys = plsc.load_gather(xbp, [rows, cols, re1])
                    o = a * y + b * ys
                    plsc.store_scatter(obp, [rows, ocols, re0], o)
                return carry2

            lax.fori_loop(0, NGRP, grp_body, 0, unroll=False)

            out_copy(c, par).start()

            @pl.when(c + 2 < NCH)
            def _():
                in_copy(c + 2, par).start()
        return carry

    lax.fori_loop(0, NCH // 2, chunk_pair, 0, unroll=False)

    out_copy(NCH - 2, 0).wait()
    out_copy(NCH - 1, 1).wait()


def kernel(x, forward_indices, crossings):
    del forward_indices
    cr = crossings[:, 0]
    ci = crossings[:, 1]
    a = jnp.stack([cr, cr], axis=-1).reshape(WIDTH)
    b = jnp.stack([-ci, ci], axis=-1).reshape(WIDTH)

    run = pl.kernel(
        _sc_body,
        out_type=jax.ShapeDtypeStruct((BATCH, LENGTH, 2), jnp.float32),
        mesh=plsc.VectorSubcoreMesh(core_axis_name="c", subcore_axis_name="s"),
        compiler_params=pltpu.CompilerParams(
            use_tc_tiling_on_sc=False, needs_layout_passes=False),
        scratch_types=[
            pltpu.VMEM((2, CH, LENGTH, 2), jnp.float32),
            pltpu.VMEM((2, CH, LENGTH, 2), jnp.float32),
            pltpu.VMEM((WIDTH,), jnp.float32),
            pltpu.VMEM((WIDTH,), jnp.float32),
            pltpu.SemaphoreType.DMA,
            pltpu.SemaphoreType.DMA,
            pltpu.SemaphoreType.DMA,
            pltpu.SemaphoreType.DMA,
        ],
    )
    return run(x, a, b)


# speedup vs baseline: 170.6849x; 170.6849x over previous
import jax
import jax.numpy as jnp
from jax import lax
from jax.experimental import pallas as pl
from jax.experimental.pallas import tpu as pltpu
from jax.experimental.pallas import tpu_sc as plsc

BATCH = 16384
LENGTH = 1024
NBLK = 8
NW = 32
RPW = BATCH // NW
CH = 8
NCH = RPW // CH


def _sc_body(z_hbm, cr_hbm, ci_hbm, o_hbm, xb, ob, crv, civ,
             isem0, isem1, osem0, osem1):
    wid = lax.axis_index("s") * 2 + lax.axis_index("c")
    base_row = wid * RPW

    pltpu.sync_copy(cr_hbm, crv)
    pltpu.sync_copy(ci_hbm, civ)

    lane = lax.iota(jnp.int32, 16)
    m4 = lane & 3
    cpat = lane + jnp.where(m4 == 1, 1, 0) - jnp.where(m4 == 2, 1, 0)

    isems = (isem0, isem1)
    osems = (osem0, osem1)

    def in_copy(c, par):
        row0 = base_row + c * CH
        return pltpu.make_async_copy(
            z_hbm.at[pl.ds(row0, CH)], xb.at[par], isems[par])

    def out_copy(c, par):
        row0 = base_row + c * CH
        return pltpu.make_async_copy(
            ob.at[par], o_hbm.at[pl.ds(row0, CH)], osems[par])

    in_copy(0, 0).start()
    in_copy(1, 1).start()

    def chunk_pair(c2, carry):
        for par in (0, 1):
            c = 2 * c2 + par
            in_copy(c, par).wait()

            @pl.when(c >= 2)
            def _():
                out_copy(c - 2, par).wait()

            xbp = xb.at[par]
            obp = ob.at[par]

            def blk_body(k, carry2):
                for g in range(8):
                    c0 = g * 16
                    a = crv[pl.ds(k * 128 + c0, 16)]
                    b = civ[pl.ds(k * 128 + c0, 16)]
                    for r in range(CH):
                        xre = xbp[r, 2 * k, pl.ds(c0, 16)]
                        xim = xbp[r, 2 * k + 1, pl.ds(c0, 16)]
                        yre = xre.at[cpat].get(mode="promise_in_bounds")
                        yim = xim.at[cpat].get(mode="promise_in_bounds")
                        ore = a * yre - b * yim
                        oim = a * yim + b * yre
                        obp[r, 2 * k, pl.ds(c0, 16)] = ore
                        obp[r, 2 * k + 1, pl.ds(c0, 16)] = oim
                return carry2

            lax.fori_loop(0, NBLK, blk_body, 0, unroll=False)

            out_copy(c, par).start()

            @pl.when(c + 2 < NCH)
            def _():
                in_copy(c + 2, par).start()
        return carry

    lax.fori_loop(0, NCH // 2, chunk_pair, 0, unroll=False)

    out_copy(NCH - 2, 0).wait()
    out_copy(NCH - 1, 1).wait()


def kernel(x, forward_indices, crossings):
    del forward_indices
    z = x.reshape(BATCH, NBLK, 128, 2).transpose(0, 1, 3, 2).reshape(
        BATCH, 2 * NBLK, 128)
    cr = crossings[:, 0]
    ci = crossings[:, 1]

    run = pl.kernel(
        _sc_body,
        out_type=jax.ShapeDtypeStruct((BATCH, 2 * NBLK, 128), jnp.float32),
        mesh=plsc.VectorSubcoreMesh(core_axis_name="c", subcore_axis_name="s"),
        compiler_params=pltpu.CompilerParams(
            use_tc_tiling_on_sc=False, needs_layout_passes=False),
        scratch_types=[
            pltpu.VMEM((2, CH, 2 * NBLK, 128), jnp.float32),
            pltpu.VMEM((2, CH, 2 * NBLK, 128), jnp.float32),
            pltpu.VMEM((LENGTH,), jnp.float32),
            pltpu.VMEM((LENGTH,), jnp.float32),
            pltpu.SemaphoreType.DMA,
            pltpu.SemaphoreType.DMA,
            pltpu.SemaphoreType.DMA,
            pltpu.SemaphoreType.DMA,
        ],
    )
    oz = run(z, cr, ci)
    return oz.reshape(BATCH, NBLK, 2, 128).transpose(0, 1, 3, 2).reshape(
        BATCH, LENGTH, 2)
